# Initial kernel scaffold; baseline (speedup 1.0000x reference)
import jax
import jax.numpy as jnp
from jax.experimental import pallas as pl


def kernel(input_ids, word_emb, pos_emb, tok_emb, ln_gamma, ln_beta):
    raise NotImplementedError("write your pallas kernel here")


# speedup vs baseline: 2.3484x; 2.3484x over previous
import functools

import jax
import jax.numpy as jnp
from jax import lax
from jax.experimental import pallas as pl
from jax.experimental.pallas import tpu as pltpu
from jax.experimental.pallas import tpu_sc as plsc

_EPS = 1e-12
_LANES = 16
_NWORKERS = 32
_NCORES = 2


@functools.cache
def _build(B, S, H, V):
    assert H % _LANES == 0
    NJ = H // _LANES
    assert B % _NWORKERS == 0
    seq_per_w = B // _NWORKERS
    half = S // 2
    assert 2 * half == S and half <= 128
    inv_h = 1.0 / H

    mesh = plsc.VectorSubcoreMesh(core_axis_name="c", subcore_axis_name="s")

    def body(ids_hbm, word_hbm, pos_hbm, tok_hbm, gam_hbm, bet_hbm, out_hbm,
             idx_v, buf_v, comb_v, gam_v, bet_v, tok_v, sem):
        wid = lax.axis_index("s") * _NCORES + lax.axis_index("c")

        pltpu.sync_copy(pos_hbm.at[pl.ds(0, S)], comb_v)
        pltpu.sync_copy(tok_hbm.at[0], tok_v)
        pltpu.sync_copy(gam_hbm, gam_v)
        pltpu.sync_copy(bet_hbm, bet_v)

        def add_tok(r, carry):
            for j in range(NJ):
                sl = pl.ds(j * _LANES, _LANES)
                comb_v[r, sl] = comb_v[r, sl] + tok_v[sl]
            return carry

        lax.fori_loop(0, S, add_tok, 0)

        def per_seq(g, carry):
            seq = wid * seq_per_w + g
            pltpu.sync_copy(ids_hbm.at[seq], idx_v)
            cp0 = pltpu.async_copy(word_hbm.at[idx_v.at[0]],
                                   buf_v.at[pl.ds(0, half)], sem)
            cp1 = pltpu.async_copy(word_hbm.at[idx_v.at[1]],
                                   buf_v.at[pl.ds(half, half)], sem)
            cp0.wait()
            cp1.wait()

            def row(r, c2):
                x = []
                for j in range(NJ):
                    sl = pl.ds(j * _LANES, _LANES)
                    x.append(buf_v[r, sl] + comb_v[r, sl])
                s1 = x[0]
                for j in range(1, NJ):
                    s1 = s1 + x[j]
                s2 = x[0] * x[0]
                for j in range(1, NJ):
                    s2 = s2 + x[j] * x[j]
                tot1 = plsc.cumsum(s1)[_LANES - 1]
                tot2 = plsc.cumsum(s2)[_LANES - 1]
                mean = tot1 * inv_h
                var = tot2 * inv_h - mean * mean
                a = var + _EPS
                i = lax.bitcast_convert_type(a, jnp.int32)
                i = 0x5F3759DF - lax.shift_right_arithmetic(i, 1)
                y = lax.bitcast_convert_type(i, jnp.float32)
                ah = 0.5 * a
                for _ in range(3):
                    y = y * (1.5 - ah * y * y)
                for j in range(NJ):
                    sl = pl.ds(j * _LANES, _LANES)
                    o = (x[j] - mean) * y * gam_v[sl] + bet_v[sl]
                    buf_v[r, sl] = o
                return c2

            lax.fori_loop(0, S, row, 0)
            pltpu.sync_copy(buf_v, out_hbm.at[pl.ds(seq * S, S)])
            return carry

        lax.fori_loop(0, seq_per_w, per_seq, 0)

    return pl.kernel(
        body,
        out_type=jax.ShapeDtypeStruct((B * S, H), jnp.float32),
        mesh=mesh,
        compiler_params=pltpu.CompilerParams(needs_layout_passes=False),
        scratch_types=[
            pltpu.VMEM((2, half), jnp.int32),
            pltpu.VMEM((S, H), jnp.float32),
            pltpu.VMEM((S, H), jnp.float32),
            pltpu.VMEM((H,), jnp.float32),
            pltpu.VMEM((H,), jnp.float32),
            pltpu.VMEM((H,), jnp.float32),
            pltpu.SemaphoreType.DMA,
        ],
    )


def kernel(input_ids, word_emb, pos_emb, tok_emb, ln_gamma, ln_beta):
    B, S = input_ids.shape
    V, H = word_emb.shape
    ids3 = input_ids.astype(jnp.int32).reshape(B, 2, S // 2)
    out_flat = _build(B, S, H, V)(ids3, word_emb, pos_emb, tok_emb,
                                  ln_gamma, ln_beta)
    return out_flat.reshape(B, S, H)


# speedup vs baseline: 10.7493x; 4.5773x over previous
import functools

import jax
import jax.numpy as jnp
from jax import lax
from jax.experimental import pallas as pl
from jax.experimental.pallas import tpu as pltpu
from jax.experimental.pallas import tpu_sc as plsc

_EPS = 1e-12
_LANES = 16
_NWORKERS = 32
_NCORES = 2
_NBUF = 4


@functools.cache
def _build(B, S, H, V):
    assert H % _LANES == 0
    NJ = H // _LANES
    assert B % _NWORKERS == 0
    spw = B // _NWORKERS
    assert spw % _NBUF == 0 and spw >= 2 * _NBUF
    half = S // 2
    assert 2 * half == S and half <= 128
    inv_h = 1.0 / H

    mesh = plsc.VectorSubcoreMesh(core_axis_name="c", subcore_axis_name="s")

    def body(ids_hbm, word_hbm, pos_hbm, tok_hbm, gam_hbm, bet_hbm, out_hbm,
             idx_v, buf_v, comb_v, gam_v, bet_v, tok_v, *sems):
        gsems = sems[:_NBUF]
        osems = sems[_NBUF:]
        wid = lax.axis_index("s") * _NCORES + lax.axis_index("c")
        base = wid * spw

        pltpu.sync_copy(pos_hbm.at[pl.ds(0, S)], comb_v)
        pltpu.sync_copy(tok_hbm.at[0], tok_v)
        pltpu.sync_copy(gam_hbm, gam_v)
        pltpu.sync_copy(bet_hbm, bet_v)

        @plsc.parallel_loop(0, S)
        def _(r):
            for j in range(NJ):
                sl = pl.ds(j * _LANES, _LANES)
                comb_v[r, sl] = comb_v[r, sl] + tok_v[sl]

        gvec = [gam_v[pl.ds(j * _LANES, _LANES)] for j in range(NJ)]
        bvec = [bet_v[pl.ds(j * _LANES, _LANES)] for j in range(NJ)]

        def issue_gather(slot, seq):
            pltpu.sync_copy(ids_hbm.at[seq], idx_v.at[slot])
            pltpu.async_copy(word_hbm.at[idx_v.at[slot, 0]],
                             buf_v.at[slot, pl.ds(0, half)], gsems[slot])
            pltpu.async_copy(word_hbm.at[idx_v.at[slot, 1]],
                             buf_v.at[slot, pl.ds(half, half)], gsems[slot])

        def wait_gather(slot):
            pltpu.make_async_copy(word_hbm.at[pl.ds(0, S)],
                                  buf_v.at[slot], gsems[slot]).wait()

        def issue_scatter(slot, seq):
            pltpu.async_copy(buf_v.at[slot],
                             out_hbm.at[pl.ds(seq * S, S)], osems[slot])

        def wait_scatter(slot):
            pltpu.make_async_copy(buf_v.at[slot],
                                  out_hbm.at[pl.ds(0, S)], osems[slot]).wait()

        def compute(slot):
            @plsc.parallel_loop(0, S, unroll=2)
            def _(r):
                x = []
                for j in range(NJ):
                    sl = pl.ds(j * _LANES, _LANES)
                    x.append(buf_v[slot, r, sl] + comb_v[r, sl])
                s1 = x[0]
                for j in range(1, NJ):
                    s1 = s1 + x[j]
                s2 = x[0] * x[0]
                for j in range(1, NJ):
                    s2 = s2 + x[j] * x[j]
                tot1 = plsc.cumsum(s1)[_LANES - 1]
                tot2 = plsc.cumsum(s2)[_LANES - 1]
                mean = tot1 * inv_h
                var = tot2 * inv_h - mean * mean
                a = var + _EPS
                i = lax.bitcast_convert_type(a, jnp.int32)
                i = 0x5F3759DF - lax.shift_right_arithmetic(i, 1)
                y = lax.bitcast_convert_type(i, jnp.float32)
                ah = 0.5 * a
                for _ in range(3):
                    y = y * (1.5 - ah * y * y)
                for j in range(NJ):
                    sl = pl.ds(j * _LANES, _LANES)
                    buf_v[slot, r, sl] = (x[j] - mean) * y * gvec[j] + bvec[j]

        issue_gather(0, base)

        def group(gg, carry):
            for slot in range(_NBUF):
                g = gg * _NBUF + slot
                seq = base + g
                nslot = (slot + 1) % _NBUF

                @pl.when(g + 1 < spw)
                def _():
                    @pl.when(g + 1 >= _NBUF)
                    def _():
                        wait_scatter(nslot)
                    issue_gather(nslot, seq + 1)

                wait_gather(slot)
                compute(slot)
                issue_scatter(slot, seq)
            return carry

        lax.fori_loop(0, spw // _NBUF, group, 0)
        for slot in range(_NBUF):
            wait_scatter(slot)

    return pl.kernel(
        body,
        out_type=jax.ShapeDtypeStruct((B * S, H), jnp.float32),
        mesh=mesh,
        compiler_params=pltpu.CompilerParams(needs_layout_passes=False),
        scratch_types=[
            pltpu.VMEM((_NBUF, 2, half), jnp.int32),
            pltpu.VMEM((_NBUF, S, H), jnp.float32),
            pltpu.VMEM((S, H), jnp.float32),
            pltpu.VMEM((H,), jnp.float32),
            pltpu.VMEM((H,), jnp.float32),
            pltpu.VMEM((H,), jnp.float32),
        ] + [pltpu.SemaphoreType.DMA] * (2 * _NBUF),
    )


def kernel(input_ids, word_emb, pos_emb, tok_emb, ln_gamma, ln_beta):
    B, S = input_ids.shape
    V, H = word_emb.shape
    ids3 = input_ids.astype(jnp.int32).reshape(B, 2, S // 2)
    out_flat = _build(B, S, H, V)(ids3, word_emb, pos_emb, tok_emb,
                                  ln_gamma, ln_beta)
    return out_flat.reshape(B, S, H)
